# baseline (speedup 1.0000x reference)
import functools

import jax
import jax.numpy as jnp
from jax.experimental import pallas as pl
from jax.experimental.pallas import tpu as pltpu
from jax.experimental.pallas import tpu_sc as plsc

DIM = 1024
E = 16
H = 8
HD = DIM // H
SCALE = HD ** -0.5
SWITCHLOSS = 0.1
ZLOSS = 0.001
B = 2
N = 2048
T = B * N

BT = 512
BQ = 256
NW = 32
TOKW = T // NW


def _logits_kv_kernel(x_ref, wg_ref, wkv_ref, bkv_ref,
                      logits_ref, k_ref, v_ref, zstat_ref):
    i = pl.program_id(0)
    xb = x_ref[...]

    logits = jnp.dot(xb, wg_ref[...], preferred_element_type=jnp.float32)
    logits_ref[...] = logits
    m = jnp.max(logits, axis=1, keepdims=True)
    se = jnp.sum(jnp.exp(logits - m), axis=1, keepdims=True)
    lse = jnp.log(se) + m
    zacc = jnp.sum(lse * lse)
    zrow = jnp.full((8, E), zacc, dtype=jnp.float32)

    kv = jnp.dot(xb.astype(jnp.bfloat16), wkv_ref[...],
                 preferred_element_type=jnp.float32)
    kv = kv + bkv_ref[...]
    k_ref[...] = kv[:, :HD].astype(jnp.bfloat16)
    v_ref[...] = jnp.concatenate(
        [kv[:, HD:].astype(jnp.bfloat16),
         jnp.ones((BT, HD), jnp.bfloat16)], axis=1)

    @pl.when(i == 0)
    def _init():
        zstat_ref[...] = zrow

    @pl.when(i > 0)
    def _acc():
        zstat_ref[...] = zstat_ref[...] + zrow


def _router_kernel(logits_hbm, g_hbm, idx_hbm, freq_hbm, psum_hbm,
                   lvm, gvm, ivm, fvm, pvm):
    c = jax.lax.axis_index("c")
    s_ = jax.lax.axis_index("s")
    wid = s_ * 2 + c
    base = wid * TOKW
    pltpu.sync_copy(logits_hbm.at[pl.ds(base, TOKW)], lvm)

    iota16 = jax.lax.iota(jnp.int32, 16)
    mask8 = iota16 < H
    fvm[...] = jnp.zeros((E,), jnp.float32)
    ones16 = jnp.ones((E,), jnp.float32)

    def body(i, psacc):
        row = lvm[i]
        mx = jnp.max(row)
        ex = jnp.exp(row - mx)
        probs = ex / jnp.sum(ex)
        sp, si = plsc.sort_key_val(probs, iota16, descending=True)
        topg = jnp.where(mask8, sp, 0.0)
        gn = topg / (jnp.sum(topg) + 1e-6)
        gvm[i] = gn
        ivm[i] = si
        plsc.addupdate_scatter(fvm, [si], ones16, mask=mask8)
        return psacc + probs

    ps = jax.lax.fori_loop(0, TOKW, body, jnp.zeros((E,), jnp.float32))
    pvm[...] = ps
    pltpu.sync_copy(gvm, g_hbm.at[pl.ds(base, TOKW)])
    pltpu.sync_copy(ivm, idx_hbm.at[pl.ds(base, TOKW)])
    pltpu.sync_copy(fvm, freq_hbm.at[wid])
    pltpu.sync_copy(pvm, psum_hbm.at[wid])


def _qsel_kernel(x_ref, wqt_ref, idx_ref, q_ref):
    xb16 = x_ref[...].astype(jnp.bfloat16)
    allq = jnp.dot(xb16, wqt_ref[...], preferred_element_type=jnp.float32)
    allq16 = (allq * (SCALE * 1.4426950408889634)).astype(jnp.bfloat16)
    slots = [allq16[:, e * HD:(e + 1) * HD] for e in range(E)]
    idx = idx_ref[...]
    for k in range(H):
        idxk = idx[:, k:k + 1]
        lvl = slots
        for bit in range(4):
            m_ = ((idxk >> bit) & 1) == 1
            lvl = [jnp.where(m_, lvl[2 * j + 1], lvl[2 * j])
                   for j in range(len(lvl) // 2)]
        q_ref[k, :, :] = lvl[0]


def _attn_combine_kernel(q_ref, k_ref, v_ref, g_ref, idx_ref, wout_ref,
                         zstat_ref, freq_ref, psum_ref, y_ref, aux_ref):
    b = pl.program_id(0)
    i = pl.program_id(1)

    kk = k_ref[...]
    vv = v_ref[...]
    g = g_ref[...]
    idx = idx_ref[...]
    zero = jnp.zeros((BQ, HD), jnp.bfloat16)
    xe = [zero] * E
    for h in range(H):
        s = jax.lax.dot_general(q_ref[h], kk, (((1,), (1,)), ((), ())),
                                preferred_element_type=jnp.float32)
        p = jnp.exp2(s.astype(jnp.bfloat16))
        oe = jnp.dot(p, vv, preferred_element_type=jnp.float32)
        o = oe[:, :HD] / oe[:, HD:]
        go = (g[:, h:h + 1] * o).astype(jnp.bfloat16)
        idxh = idx[:, h:h + 1]
        for e in range(E):
            xe[e] = xe[e] + jnp.where(idxh == e, go, zero)

    xef = jnp.concatenate(xe, axis=1)
    y_ref[...] = jnp.dot(xef, wout_ref[...],
                         preferred_element_type=jnp.float32)

    @pl.when((b == 0) & (i == 0))
    def _aux():
        freqs = jnp.sum(freq_ref[...], axis=0, keepdims=True)
        p_sum = jnp.sum(psum_ref[...], axis=0, keepdims=True)
        zacc = jnp.sum(zstat_ref[0:1, 0:1])
        norm_p = p_sum / (jnp.sum(jnp.abs(p_sum)) + 1e-12)
        norm_f = freqs / (jnp.sum(jnp.abs(freqs)) + 1e-12)
        switch = E * jnp.sum(norm_p * norm_f)
        zl = zacc / T
        aux_ref[...] = jnp.full((1, 1), SWITCHLOSS * switch + ZLOSS * zl,
                                dtype=jnp.float32)


@jax.jit
def kernel(x, Wg, Wq, Wout, Wkv, bkv):
    xf = x.reshape(T, DIM)
    wqt = Wq.transpose(1, 0, 2).reshape(DIM, E * HD).astype(jnp.bfloat16)
    wkv16 = Wkv.astype(jnp.bfloat16)
    wout_f = Wout.reshape(E * HD, DIM).astype(jnp.bfloat16)
    bkv2 = bkv.reshape(1, 2 * HD)

    nt = T // BT
    logits, k, v, zstat = pl.pallas_call(
        _logits_kv_kernel,
        grid=(nt,),
        in_specs=[
            pl.BlockSpec((BT, DIM), lambda i: (i, 0)),
            pl.BlockSpec((DIM, E), lambda i: (0, 0)),
            pl.BlockSpec((DIM, 2 * HD), lambda i: (0, 0)),
            pl.BlockSpec((1, 2 * HD), lambda i: (0, 0)),
        ],
        out_specs=[
            pl.BlockSpec((BT, E), lambda i: (i, 0)),
            pl.BlockSpec((BT, HD), lambda i: (i, 0)),
            pl.BlockSpec((BT, 2 * HD), lambda i: (i, 0)),
            pl.BlockSpec((8, E), lambda i: (0, 0)),
        ],
        out_shape=[
            jax.ShapeDtypeStruct((T, E), jnp.float32),
            jax.ShapeDtypeStruct((T, HD), jnp.bfloat16),
            jax.ShapeDtypeStruct((T, 2 * HD), jnp.bfloat16),
            jax.ShapeDtypeStruct((8, E), jnp.float32),
        ],
    )(xf, Wg, wkv16, bkv2)

    router = functools.partial(
        pl.kernel,
        mesh=plsc.VectorSubcoreMesh(core_axis_name="c", subcore_axis_name="s"),
        compiler_params=pltpu.CompilerParams(needs_layout_passes=False),
        out_type=[
            jax.ShapeDtypeStruct((T, E), jnp.float32),
            jax.ShapeDtypeStruct((T, E), jnp.int32),
            jax.ShapeDtypeStruct((NW, E), jnp.float32),
            jax.ShapeDtypeStruct((NW, E), jnp.float32),
        ],
        scratch_types=[
            pltpu.VMEM((TOKW, E), jnp.float32),
            pltpu.VMEM((TOKW, E), jnp.float32),
            pltpu.VMEM((TOKW, E), jnp.int32),
            pltpu.VMEM((E,), jnp.float32),
            pltpu.VMEM((E,), jnp.float32),
        ],
    )(_router_kernel)
    g16, idx16, freqp, psump = router(logits)

    q = pl.pallas_call(
        _qsel_kernel,
        grid=(nt,),
        in_specs=[
            pl.BlockSpec((BT, DIM), lambda i: (i, 0)),
            pl.BlockSpec((DIM, E * HD), lambda i: (0, 0)),
            pl.BlockSpec((BT, E), lambda i: (i, 0)),
        ],
        out_specs=pl.BlockSpec((H, BT, HD), lambda i: (0, i, 0)),
        out_shape=jax.ShapeDtypeStruct((H, T, HD), jnp.bfloat16),
    )(xf, wqt, idx16)

    nb = N // BQ
    y, aux = pl.pallas_call(
        _attn_combine_kernel,
        grid=(B, nb),
        in_specs=[
            pl.BlockSpec((H, BQ, HD), lambda b, i: (0, b * nb + i, 0)),
            pl.BlockSpec((N, HD), lambda b, i: (b, 0)),
            pl.BlockSpec((N, 2 * HD), lambda b, i: (b, 0)),
            pl.BlockSpec((BQ, E), lambda b, i: (b * nb + i, 0)),
            pl.BlockSpec((BQ, E), lambda b, i: (b * nb + i, 0)),
            pl.BlockSpec((E * HD, DIM), lambda b, i: (0, 0)),
            pl.BlockSpec((8, E), lambda b, i: (0, 0)),
            pl.BlockSpec((NW, E), lambda b, i: (0, 0)),
            pl.BlockSpec((NW, E), lambda b, i: (0, 0)),
        ],
        out_specs=[
            pl.BlockSpec((BQ, DIM), lambda b, i: (b * nb + i, 0)),
            pl.BlockSpec((1, 1), lambda b, i: (0, 0)),
        ],
        out_shape=[
            jax.ShapeDtypeStruct((T, DIM), jnp.float32),
            jax.ShapeDtypeStruct((1, 1), jnp.float32),
        ],
    )(q, k, v, g16, idx16, wout_f, zstat, freqp, psump)

    return y.reshape(B, N, DIM), aux[0, 0]


# speedup vs baseline: 1.0479x; 1.0479x over previous
import functools

import jax
import jax.numpy as jnp
from jax.experimental import pallas as pl
from jax.experimental.pallas import tpu as pltpu
from jax.experimental.pallas import tpu_sc as plsc

DIM = 1024
E = 16
H = 8
HD = DIM // H
SCALE = HD ** -0.5
SWITCHLOSS = 0.1
ZLOSS = 0.001
B = 2
N = 2048
T = B * N

BT = 1024
BQ = 512
NW = 32
TOKW = T // NW


def _logits_kv_kernel(x_ref, wg_ref, wkv_ref, bkv_ref,
                      logits_ref, k_ref, v_ref, zstat_ref):
    i = pl.program_id(0)
    xb = x_ref[...]

    logits = jnp.dot(xb, wg_ref[...], preferred_element_type=jnp.float32)
    logits_ref[...] = logits
    m = jnp.max(logits, axis=1, keepdims=True)
    se = jnp.sum(jnp.exp(logits - m), axis=1, keepdims=True)
    lse = jnp.log(se) + m
    zacc = jnp.sum(lse * lse)
    zrow = jnp.full((8, E), zacc, dtype=jnp.float32)

    kv = jnp.dot(xb.astype(jnp.bfloat16), wkv_ref[...],
                 preferred_element_type=jnp.float32)
    kv = kv + bkv_ref[...]
    k_ref[...] = kv[:, :HD].astype(jnp.bfloat16)
    v_ref[...] = jnp.concatenate(
        [kv[:, HD:].astype(jnp.bfloat16),
         jnp.ones((BT, HD), jnp.bfloat16)], axis=1)

    @pl.when(i == 0)
    def _init():
        zstat_ref[...] = zrow

    @pl.when(i > 0)
    def _acc():
        zstat_ref[...] = zstat_ref[...] + zrow


def _router_kernel(logits_hbm, g_hbm, idx_hbm, freq_hbm, psum_hbm,
                   lvm, gvm, ivm, fvm, pvm):
    c = jax.lax.axis_index("c")
    s_ = jax.lax.axis_index("s")
    wid = s_ * 2 + c
    base = wid * TOKW
    pltpu.sync_copy(logits_hbm.at[pl.ds(base, TOKW)], lvm)

    iota16 = jax.lax.iota(jnp.int32, 16)
    mask8 = iota16 < H
    fvm[...] = jnp.zeros((E,), jnp.float32)
    ones16 = jnp.ones((E,), jnp.float32)

    def body(i, psacc):
        row = lvm[i]
        mx = jnp.max(row)
        ex = jnp.exp(row - mx)
        probs = ex / jnp.sum(ex)
        sp, si = plsc.sort_key_val(probs, iota16, descending=True)
        topg = jnp.where(mask8, sp, 0.0)
        gn = topg / (jnp.sum(topg) + 1e-6)
        gvm[i] = gn
        ivm[i] = si
        plsc.addupdate_scatter(fvm, [si], ones16, mask=mask8)
        return psacc + probs

    ps = jax.lax.fori_loop(0, TOKW, body, jnp.zeros((E,), jnp.float32))
    pvm[...] = ps
    pltpu.sync_copy(gvm, g_hbm.at[pl.ds(base, TOKW)])
    pltpu.sync_copy(ivm, idx_hbm.at[pl.ds(base, TOKW)])
    pltpu.sync_copy(fvm, freq_hbm.at[wid])
    pltpu.sync_copy(pvm, psum_hbm.at[wid])


def _qsel_kernel(x_ref, wqt_ref, idx_ref, q_ref):
    xb16 = x_ref[...].astype(jnp.bfloat16)
    allq = jnp.dot(xb16, wqt_ref[...], preferred_element_type=jnp.float32)
    allq16 = (allq * (SCALE * 1.4426950408889634)).astype(jnp.bfloat16)
    slots = [allq16[:, e * HD:(e + 1) * HD] for e in range(E)]
    idx = idx_ref[...]
    for k in range(H):
        idxk = idx[:, k:k + 1]
        lvl = slots
        for bit in range(4):
            m_ = ((idxk >> bit) & 1) == 1
            lvl = [jnp.where(m_, lvl[2 * j + 1], lvl[2 * j])
                   for j in range(len(lvl) // 2)]
        q_ref[k, :, :] = lvl[0]


def _attn_combine_kernel(q_ref, k_ref, v_ref, g_ref, idx_ref, wout_ref,
                         zstat_ref, freq_ref, psum_ref, y_ref, aux_ref):
    b = pl.program_id(0)
    i = pl.program_id(1)

    kk = k_ref[...]
    vv = v_ref[...]
    g = g_ref[...]
    idx = idx_ref[...]
    zero = jnp.zeros((BQ, HD), jnp.bfloat16)
    xe = [zero] * E
    for h in range(H):
        s = jax.lax.dot_general(q_ref[h], kk, (((1,), (1,)), ((), ())),
                                preferred_element_type=jnp.float32)
        p = jnp.exp2(s.astype(jnp.bfloat16))
        oe = jnp.dot(p, vv, preferred_element_type=jnp.float32)
        o = oe[:, :HD] / oe[:, HD:]
        go = (g[:, h:h + 1] * o).astype(jnp.bfloat16)
        idxh = idx[:, h:h + 1]
        for e in range(E):
            xe[e] = xe[e] + jnp.where(idxh == e, go, zero)

    xef = jnp.concatenate(xe, axis=1)
    y_ref[...] = jnp.dot(xef, wout_ref[...],
                         preferred_element_type=jnp.float32)

    @pl.when((b == 0) & (i == 0))
    def _aux():
        freqs = jnp.sum(freq_ref[...], axis=0, keepdims=True)
        p_sum = jnp.sum(psum_ref[...], axis=0, keepdims=True)
        zacc = jnp.sum(zstat_ref[0:1, 0:1])
        norm_p = p_sum / (jnp.sum(jnp.abs(p_sum)) + 1e-12)
        norm_f = freqs / (jnp.sum(jnp.abs(freqs)) + 1e-12)
        switch = E * jnp.sum(norm_p * norm_f)
        zl = zacc / T
        aux_ref[...] = jnp.full((1, 1), SWITCHLOSS * switch + ZLOSS * zl,
                                dtype=jnp.float32)


@jax.jit
def kernel(x, Wg, Wq, Wout, Wkv, bkv):
    xf = x.reshape(T, DIM)
    wqt = Wq.transpose(1, 0, 2).reshape(DIM, E * HD).astype(jnp.bfloat16)
    wkv16 = Wkv.astype(jnp.bfloat16)
    wout_f = Wout.reshape(E * HD, DIM).astype(jnp.bfloat16)
    bkv2 = bkv.reshape(1, 2 * HD)

    nt = T // BT
    logits, k, v, zstat = pl.pallas_call(
        _logits_kv_kernel,
        grid=(nt,),
        in_specs=[
            pl.BlockSpec((BT, DIM), lambda i: (i, 0)),
            pl.BlockSpec((DIM, E), lambda i: (0, 0)),
            pl.BlockSpec((DIM, 2 * HD), lambda i: (0, 0)),
            pl.BlockSpec((1, 2 * HD), lambda i: (0, 0)),
        ],
        out_specs=[
            pl.BlockSpec((BT, E), lambda i: (i, 0)),
            pl.BlockSpec((BT, HD), lambda i: (i, 0)),
            pl.BlockSpec((BT, 2 * HD), lambda i: (i, 0)),
            pl.BlockSpec((8, E), lambda i: (0, 0)),
        ],
        out_shape=[
            jax.ShapeDtypeStruct((T, E), jnp.float32),
            jax.ShapeDtypeStruct((T, HD), jnp.bfloat16),
            jax.ShapeDtypeStruct((T, 2 * HD), jnp.bfloat16),
            jax.ShapeDtypeStruct((8, E), jnp.float32),
        ],
    )(xf, Wg, wkv16, bkv2)

    router = functools.partial(
        pl.kernel,
        mesh=plsc.VectorSubcoreMesh(core_axis_name="c", subcore_axis_name="s"),
        compiler_params=pltpu.CompilerParams(needs_layout_passes=False),
        out_type=[
            jax.ShapeDtypeStruct((T, E), jnp.float32),
            jax.ShapeDtypeStruct((T, E), jnp.int32),
            jax.ShapeDtypeStruct((NW, E), jnp.float32),
            jax.ShapeDtypeStruct((NW, E), jnp.float32),
        ],
        scratch_types=[
            pltpu.VMEM((TOKW, E), jnp.float32),
            pltpu.VMEM((TOKW, E), jnp.float32),
            pltpu.VMEM((TOKW, E), jnp.int32),
            pltpu.VMEM((E,), jnp.float32),
            pltpu.VMEM((E,), jnp.float32),
        ],
    )(_router_kernel)
    g16, idx16, freqp, psump = router(logits)

    q = pl.pallas_call(
        _qsel_kernel,
        grid=(nt,),
        in_specs=[
            pl.BlockSpec((BT, DIM), lambda i: (i, 0)),
            pl.BlockSpec((DIM, E * HD), lambda i: (0, 0)),
            pl.BlockSpec((BT, E), lambda i: (i, 0)),
        ],
        out_specs=pl.BlockSpec((H, BT, HD), lambda i: (0, i, 0)),
        out_shape=jax.ShapeDtypeStruct((H, T, HD), jnp.bfloat16),
    )(xf, wqt, idx16)

    nb = N // BQ
    y, aux = pl.pallas_call(
        _attn_combine_kernel,
        grid=(B, nb),
        in_specs=[
            pl.BlockSpec((H, BQ, HD), lambda b, i: (0, b * nb + i, 0)),
            pl.BlockSpec((N, HD), lambda b, i: (b, 0)),
            pl.BlockSpec((N, 2 * HD), lambda b, i: (b, 0)),
            pl.BlockSpec((BQ, E), lambda b, i: (b * nb + i, 0)),
            pl.BlockSpec((BQ, E), lambda b, i: (b * nb + i, 0)),
            pl.BlockSpec((E * HD, DIM), lambda b, i: (0, 0)),
            pl.BlockSpec((8, E), lambda b, i: (0, 0)),
            pl.BlockSpec((NW, E), lambda b, i: (0, 0)),
            pl.BlockSpec((NW, E), lambda b, i: (0, 0)),
        ],
        out_specs=[
            pl.BlockSpec((BQ, DIM), lambda b, i: (b * nb + i, 0)),
            pl.BlockSpec((1, 1), lambda b, i: (0, 0)),
        ],
        out_shape=[
            jax.ShapeDtypeStruct((T, DIM), jnp.float32),
            jax.ShapeDtypeStruct((1, 1), jnp.float32),
        ],
    )(q, k, v, g16, idx16, wout_f, zstat, freqp, psump)

    return y.reshape(B, N, DIM), aux[0, 0]
